# baseline (speedup 1.0000x reference)
import functools

import jax
import jax.numpy as jnp
from jax import lax
from jax.experimental import pallas as pl
from jax.experimental.pallas import tpu as pltpu
from jax.experimental.pallas import tpu_sc as plsc

_RPW = 8
_RB = 8
_NSF = 3


def _sc_gather_sum_body(n_active, n_cores, x_hbm, emb_hbm, out_hbm,
                        idx_v, rows_v, sum_v, sem):
    wid = lax.axis_index("s") * n_cores + lax.axis_index("c")

    @pl.when(wid < n_active)
    def _():
        pltpu.sync_copy(x_hbm.at[pl.ds(wid * _RPW, _RPW)], idx_v)
        pltpu.async_copy(emb_hbm.at[idx_v], rows_v, sem).wait()
        d = rows_v.shape[1]
        for c in range(d // 16):
            acc = rows_v[0, pl.ds(c * 16, 16)]
            for r in range(1, _RPW):
                acc = acc + rows_v[r, pl.ds(c * 16, 16)]
            sum_v[pl.ds(c * 16, 16)] = acc
        pltpu.sync_copy(sum_v, out_hbm.at[wid])


def _gather_pool_sc(x, emb, n_active):
    info = plsc.get_sparse_core_info()
    n_cores = info.num_cores
    d = emb.shape[1]
    mesh = plsc.VectorSubcoreMesh(core_axis_name="c", subcore_axis_name="s")
    body = functools.partial(_sc_gather_sum_body, n_active, n_cores)
    call = pl.kernel(
        body,
        mesh=mesh,
        out_type=jax.ShapeDtypeStruct((n_active, d), jnp.float32),
        scratch_types=[
            pltpu.VMEM((_RPW,), jnp.int32),
            pltpu.VMEM((_RPW, d), jnp.float32),
            pltpu.VMEM((d,), jnp.float32),
            pltpu.SemaphoreType.DMA,
        ],
    )
    return call(x, emb)


def _tc_body2(V, L, H, KS, partials, W1, b1, w2a, w2b, w2c, w2t, b2, out, h1s):
    i = pl.program_id(0)
    Hp = h1s.shape[1]
    n_tail = H - (KS * _NSF) * _RB

    def chunk_dot(w2val, off):
        br = lax.broadcasted_iota(jnp.int32, (Hp, _RB), 0)
        bc = lax.broadcasted_iota(jnp.int32, (Hp, _RB), 1)
        sel = jnp.where(br == off + bc, 1.0, 0.0)
        h1c = jnp.dot(h1s[...], sel, preferred_element_type=jnp.float32)
        return jnp.dot(h1c, w2val, preferred_element_type=jnp.float32)

    def dots3():
        return (chunk_dot(w2a[...], (0 * KS + i) * _RB)
                + chunk_dot(w2b[...], (1 * KS + i) * _RB)
                + chunk_dot(w2c[...], (2 * KS + i) * _RB))

    @pl.when(i == 0)
    def _():
        h = jnp.sum(partials[...], axis=0, keepdims=True) * (1.0 / L)
        pre = jnp.dot(h, W1[...], preferred_element_type=jnp.float32) + b1[...]
        h1 = jnp.maximum(pre, 0.0)
        h1s[...] = jnp.concatenate(
            [h1, jnp.zeros((1, Hp - H), jnp.float32)], axis=1)
        row = lax.broadcasted_iota(jnp.int32, (_RB, V), 0)
        w2ts = jnp.where(row < n_tail, w2t[...], 0.0)
        out[...] = b2[...] + dots3() + chunk_dot(w2ts, _NSF * KS * _RB)

    @pl.when((i >= 1) & (i < KS))
    def _():
        out[...] = out[...] + dots3()

    @pl.when(i == KS)
    def _():
        z = out[...]
        m = jnp.max(z)
        lse = m + jnp.log(jnp.sum(jnp.exp(z - m)))
        out[...] = z - lse


def kernel(x, emb, W1, b1, W2, b2):
    x = x.astype(jnp.int32)
    L = x.shape[0]
    D = emb.shape[1]
    H = W1.shape[1]
    V = W2.shape[1]
    n_active = L // _RPW

    partials = _gather_pool_sc(x, emb, n_active)

    KS = 4
    TAIL = KS * _NSF
    b1v = b1.reshape(1, H)
    b2v = b2.reshape(1, V)

    def _w2_spec(s):
        return pl.BlockSpec((_RB, V), lambda i: (s * KS + jnp.minimum(i, KS - 1), 0))

    out = pl.pallas_call(
        functools.partial(_tc_body2, V, L, H, KS),
        grid=(KS + 1,),
        in_specs=[
            pl.BlockSpec((n_active, D), lambda i: (0, 0)),
            pl.BlockSpec((D, H), lambda i: (0, 0)),
            pl.BlockSpec((1, H), lambda i: (0, 0)),
        ] + [_w2_spec(s) for s in range(_NSF)]
          + [pl.BlockSpec((_RB, V), lambda i: (TAIL, 0))]
          + [pl.BlockSpec((1, V), lambda i: (0, 0))],
        out_specs=pl.BlockSpec((1, V), lambda i: (0, 0)),
        out_shape=jax.ShapeDtypeStruct((1, V), jnp.float32),
        scratch_shapes=[pltpu.VMEM((1, 104), jnp.float32)],
    )(partials, W1, b1v, W2, W2, W2, W2, b2v)
    return out
